# baseline (speedup 1.0000x reference)
import functools

import jax
import jax.numpy as jnp
from jax.experimental import pallas as pl
from jax.experimental.pallas import tpu as pltpu


def _cosine_rows_kernel(x_ref, out_ref, *, tile: int):
    i = pl.program_id(1)
    xb = x_ref[0]
    xi = x_ref[0, :, pl.ds(i * tile, tile)]

    inv_b = jax.lax.rsqrt(jnp.maximum(jnp.sum(xb * xb, axis=0), 1e-20))
    inv_i = jax.lax.rsqrt(jnp.maximum(jnp.sum(xi * xi, axis=0), 1e-20))
    xbn = xb * inv_b[None, :]
    xin = xi * inv_i[None, :]

    sim = jax.lax.dot_general(
        xin, xbn,
        dimension_numbers=(((0,), (0,)), ((), ())),
        preferred_element_type=jnp.float32,
    )

    p = xb.shape[1]
    row_g = i * tile + jax.lax.broadcasted_iota(jnp.int32, (tile, p), 0)
    col_g = jax.lax.broadcasted_iota(jnp.int32, (tile, p), 1)
    out_ref[0] = jnp.where(row_g == col_g, jnp.float32(1.0), sim)


def kernel(x):
    b, c, p = x.shape
    tile = 2048
    grid = (b, p // tile)
    return pl.pallas_call(
        functools.partial(_cosine_rows_kernel, tile=tile),
        grid=grid,
        in_specs=[pl.BlockSpec((1, c, p), lambda bi, i: (bi, 0, 0))],
        out_specs=pl.BlockSpec((1, tile, p), lambda bi, i: (bi, i, 0)),
        out_shape=jax.ShapeDtypeStruct((b, p, p), jnp.float32),
        compiler_params=pltpu.CompilerParams(
            dimension_semantics=("parallel", "parallel"),
        ),
    )(x)


# speedup vs baseline: 1.0750x; 1.0750x over previous
import functools

import jax
import jax.numpy as jnp
from jax.experimental import pallas as pl
from jax.experimental.pallas import tpu as pltpu


def _cosine_rows_kernel(x_ref, out_ref, *, tile: int):
    i = pl.program_id(1)
    xb = x_ref[0]
    xi = x_ref[0, :, pl.ds(i * tile, tile)]

    inv_b = jax.lax.rsqrt(jnp.maximum(jnp.sum(xb * xb, axis=0), 1e-20))
    inv_i = jax.lax.rsqrt(jnp.maximum(jnp.sum(xi * xi, axis=0), 1e-20))
    xbn = xb * inv_b[None, :]
    xin = xi * inv_i[None, :]

    sim = jax.lax.dot_general(
        xin, xbn,
        dimension_numbers=(((0,), (0,)), ((), ())),
        preferred_element_type=jnp.float32,
    )

    p = xb.shape[1]
    row_g = i * tile + jax.lax.broadcasted_iota(jnp.int32, (tile, p), 0)
    col_g = jax.lax.broadcasted_iota(jnp.int32, (tile, p), 1)
    out_ref[0] = jnp.where(row_g == col_g, jnp.float32(1.0), sim)


def kernel(x):
    b, c, p = x.shape
    tile = 1024
    grid = (b, p // tile)
    return pl.pallas_call(
        functools.partial(_cosine_rows_kernel, tile=tile),
        grid=grid,
        in_specs=[pl.BlockSpec((1, c, p), lambda bi, i: (bi, 0, 0))],
        out_specs=pl.BlockSpec((1, tile, p), lambda bi, i: (bi, i, 0)),
        out_shape=jax.ShapeDtypeStruct((b, p, p), jnp.float32),
        compiler_params=pltpu.CompilerParams(
            dimension_semantics=("parallel", "parallel"),
        ),
    )(x)
